# Initial kernel scaffold; baseline (speedup 1.0000x reference)
import jax
import jax.numpy as jnp
from jax.experimental import pallas as pl


def kernel(x, edge_index, W1, b1, W2, b2, W3, b3, W4, b4):
    raise NotImplementedError("write your pallas kernel here")


# speedup vs baseline: 6.2467x; 6.2467x over previous
import functools

import jax
import jax.numpy as jnp
from jax import lax
from jax.experimental import pallas as pl
from jax.experimental.pallas import tpu as pltpu
from jax.experimental.pallas import tpu_sc as plsc

N = 10000
E = 320000
NC, NS = 2, 16
NW = NC * NS
CH = 128
K = 79
EW = K * CH
E_PAD = NW * EW
NB = 10240
NPAD = NB + 8
STRIPE = NB // NS
ZR = 32

_mesh = plsc.VectorSubcoreMesh(core_axis_name="c", subcore_axis_name="s")


@functools.partial(
    pl.kernel,
    out_type=(jax.ShapeDtypeStruct((NC, NB, 16), jnp.float32),
              jax.ShapeDtypeStruct((NC, NB, 16), jnp.float32)),
    mesh=_mesh,
    scratch_types=[
        pltpu.VMEM_SHARED((NPAD, 16), jnp.float32),
        pltpu.VMEM_SHARED((NPAD, 16), jnp.float32),
        pltpu.VMEM((K, CH), jnp.int32),
        pltpu.VMEM((K, CH), jnp.int32),
        pltpu.VMEM((CH, 16), jnp.float32),
        pltpu.VMEM((STRIPE, 16), jnp.float32),
    ],
    compiler_params=pltpu.CompilerParams(use_tc_tiling_on_sc=False),
)
def _deg_kernel(srcd_hbm, dstd_hbm, ones_hbm, zeros_hbm, dop_hbm, dip_hbm,
                acc_o, acc_i, src_v, dst_v, ones_v, zv):
    c = lax.axis_index("c")
    s = lax.axis_index("s")
    wid = s * NC + c
    base = s * STRIPE
    pltpu.sync_copy(zeros_hbm, zv)
    pltpu.sync_copy(zv, acc_o.at[pl.ds(base, STRIPE)])
    pltpu.sync_copy(zv, acc_i.at[pl.ds(base, STRIPE)])
    pltpu.sync_copy(ones_hbm, ones_v)
    plsc.subcore_barrier()
    pltpu.sync_copy(srcd_hbm.at[wid], src_v)
    pltpu.sync_copy(dstd_hbm.at[wid], dst_v)

    def step_o(j, carry):
        pltpu.sync_copy(ones_v, acc_o.at[src_v.at[j]], add=True)
        return carry

    def step_i(j, carry):
        pltpu.sync_copy(ones_v, acc_i.at[dst_v.at[j]], add=True)
        return carry

    lax.fori_loop(0, K, step_o, 0)
    lax.fori_loop(0, K, step_i, 0)
    plsc.subcore_barrier()
    pltpu.sync_copy(acc_o.at[pl.ds(base, STRIPE)], zv)
    pltpu.sync_copy(zv, dop_hbm.at[c, pl.ds(base, STRIPE)])
    pltpu.sync_copy(acc_i.at[pl.ds(base, STRIPE)], zv)
    pltpu.sync_copy(zv, dip_hbm.at[c, pl.ds(base, STRIPE)])


def _make_agg(F):
    @functools.partial(
        pl.kernel,
        out_type=jax.ShapeDtypeStruct((NC, NB, F), jnp.float32),
        mesh=_mesh,
        scratch_types=[
            pltpu.VMEM_SHARED((NPAD, F), jnp.float32),
            pltpu.VMEM((K, CH), jnp.int32),
            pltpu.VMEM((K, CH), jnp.int32),
            pltpu.VMEM((CH, F), jnp.float32),
            pltpu.VMEM((ZR, F), jnp.float32),
            pltpu.SemaphoreType.DMA,
        ],
        compiler_params=pltpu.CompilerParams(use_tc_tiling_on_sc=False),
    )
    def agg(u_hbm, srcg_hbm, dstd_hbm, p_hbm,
            accum, src_v, dst_v, rows_v, zbuf, sem):
        c = lax.axis_index("c")
        s = lax.axis_index("s")
        wid = s * NC + c
        base = s * STRIPE
        zv = jnp.zeros((16,), jnp.float32)
        for i in range(ZR):
            for f in range(F // 16):
                zbuf[i, pl.ds(f * 16, 16)] = zv
        for k in range(STRIPE // ZR):
            pltpu.sync_copy(zbuf, accum.at[pl.ds(base + k * ZR, ZR)])
        plsc.subcore_barrier()
        pltpu.sync_copy(srcg_hbm.at[wid], src_v)
        pltpu.sync_copy(dstd_hbm.at[wid], dst_v)

        def step(j, carry):
            pltpu.async_copy(u_hbm.at[src_v.at[j]], rows_v, sem).wait()
            pltpu.sync_copy(rows_v, accum.at[dst_v.at[j]], add=True)
            return carry

        lax.fori_loop(0, K, step, 0)
        plsc.subcore_barrier()
        for k in range(STRIPE // ZR):
            pltpu.sync_copy(accum.at[pl.ds(base + k * ZR, ZR)], zbuf)
            pltpu.sync_copy(zbuf, p_hbm.at[c, pl.ds(base + k * ZR, ZR)])

    return agg


_agg128 = _make_agg(128)
_agg64 = _make_agg(64)
_agg16 = _make_agg(16)


R = 400
_GRID = N // R


def _row_spec(f):
    return pl.BlockSpec((R, f), lambda i: (i, 0))


def _part_spec(f):
    return pl.BlockSpec((NC, R, f), lambda i: (0, i, 0))


def _full_spec(a, b):
    return pl.BlockSpec((a, b), lambda i: (0, 0))


def _tc0_body(x_ref, dop_ref, dip_ref, u1_ref, ns_ref, nd_ref):
    ns8 = lax.rsqrt(jnp.maximum(dop_ref[0] + dop_ref[1], 1.0))
    nd8 = lax.rsqrt(jnp.maximum(dip_ref[0] + dip_ref[1], 1.0))
    ns_ref[...] = ns8
    nd_ref[...] = nd8
    u1_ref[...] = x_ref[...] * ns8[:, 0:1]


_tc0 = pl.pallas_call(
    _tc0_body,
    grid=(_GRID,),
    in_specs=[_row_spec(128), _part_spec(16), _part_spec(16)],
    out_specs=(_row_spec(128), _row_spec(16), _row_spec(16)),
    out_shape=(jax.ShapeDtypeStruct((N, 128), jnp.float32),
               jax.ShapeDtypeStruct((N, 16), jnp.float32),
               jax.ShapeDtypeStruct((N, 16), jnp.float32)),
)


def _tc1_body(p_ref, ns_ref, nd_ref, w1_ref, b1_ref, w2_ref, u2_ref):
    t = (p_ref[0] + p_ref[1]) * nd_ref[...][:, 0:1]
    h = jnp.dot(t, w1_ref[...], preferred_element_type=jnp.float32) + b1_ref[...]
    h = jnp.maximum(h, 0.0) * ns_ref[...][:, 0:1]
    u2_ref[...] = jnp.dot(h, w2_ref[...], preferred_element_type=jnp.float32)


_tc1 = pl.pallas_call(
    _tc1_body,
    grid=(_GRID,),
    in_specs=[_part_spec(128), _row_spec(16), _row_spec(16),
              _full_spec(128, 128), _full_spec(1, 128), _full_spec(128, 64)],
    out_specs=_row_spec(64),
    out_shape=jax.ShapeDtypeStruct((N, 64), jnp.float32),
)


def _tc2_body(p_ref, ns_ref, nd_ref, b2_ref, w3_ref, u3_ref):
    t = (p_ref[0] + p_ref[1]) * nd_ref[...][:, 0:1] + b2_ref[...]
    h = jnp.maximum(t, 0.0) * ns_ref[...][:, 0:1]
    u3_ref[...] = jnp.dot(h, w3_ref[...], preferred_element_type=jnp.float32)


_tc2 = pl.pallas_call(
    _tc2_body,
    grid=(_GRID,),
    in_specs=[_part_spec(64), _row_spec(16), _row_spec(16),
              _full_spec(1, 64), _full_spec(64, 16)],
    out_specs=_row_spec(16),
    out_shape=jax.ShapeDtypeStruct((N, 16), jnp.float32),
)


def _tc3_body(p_ref, ns_ref, nd_ref, b3_ref, u4_ref):
    t = (p_ref[0] + p_ref[1]) * nd_ref[...][:, 0:1] + b3_ref[...]
    u4_ref[...] = jnp.maximum(t, 0.0) * ns_ref[...][:, 0:1]


_tc3 = pl.pallas_call(
    _tc3_body,
    grid=(_GRID,),
    in_specs=[_part_spec(16), _row_spec(16), _row_spec(16), _full_spec(1, 16)],
    out_specs=_row_spec(16),
    out_shape=jax.ShapeDtypeStruct((N, 16), jnp.float32),
)


def _tc4_body(p_ref, nd_ref, w4_ref, b4_ref, out_ref):
    t = (p_ref[0] + p_ref[1]) * nd_ref[...][:, 0:1]
    out_ref[...] = jnp.dot(t, w4_ref[...],
                           preferred_element_type=jnp.float32) + b4_ref[...]


_tc4 = pl.pallas_call(
    _tc4_body,
    grid=(_GRID,),
    in_specs=[_part_spec(16), _row_spec(16), _full_spec(16, 40), _full_spec(1, 40)],
    out_specs=_row_spec(40),
    out_shape=jax.ShapeDtypeStruct((N, 40), jnp.float32),
)


def kernel(x, edge_index, W1, b1, W2, b2, W3, b3, W4, b4):
    ei = edge_index.astype(jnp.int32)
    src, dst = ei[0], ei[1]
    pad = E_PAD - E
    srcg = jnp.concatenate([src, jnp.zeros((pad,), jnp.int32)]).reshape(NW, K, CH)
    srcd = jnp.concatenate([src, jnp.full((pad,), NB, jnp.int32)]).reshape(NW, K, CH)
    dstd = jnp.concatenate([dst, jnp.full((pad,), NB, jnp.int32)]).reshape(NW, K, CH)
    ones8 = jnp.ones((CH, 16), jnp.float32)
    zeros8 = jnp.zeros((STRIPE, 16), jnp.float32)

    dop, dip = _deg_kernel(srcd, dstd, ones8, zeros8)
    u1, ns8, nd8 = _tc0(x, dop, dip)
    p1 = _agg128(u1, srcg, dstd)
    u2 = _tc1(p1, ns8, nd8, W1, b1.reshape(1, -1), W2)
    p2 = _agg64(u2, srcg, dstd)
    u3 = _tc2(p2, ns8, nd8, b2.reshape(1, -1), W3)
    p3 = _agg16(u3, srcg, dstd)
    u4 = _tc3(p3, ns8, nd8, b3.reshape(1, -1))
    p4 = _agg16(u4, srcg, dstd)
    return _tc4(p4, nd8, W4, b4.reshape(1, -1))
